# Initial kernel scaffold; baseline (speedup 1.0000x reference)
import jax
import jax.numpy as jnp
from jax.experimental import pallas as pl


def kernel(x, indices):
    raise NotImplementedError("write your pallas kernel here")


# speedup vs baseline: 7.2533x; 7.2533x over previous
import functools

import jax
import jax.numpy as jnp
from jax import lax
from jax.experimental import pallas as pl
from jax.experimental.pallas import tpu as pltpu
from jax.experimental.pallas import tpu_sc as plsc

NX = 512
NY = 512
NCELLS = NX * NY
F = 6
NC = 2
NS = 16
NT = NC * NS
CHUNK = 6272
NP_PAD = NT * CHUNK
ACC = F * NCELLS
ACC_SLICE = ACC // NS
ZB = 2048

_mesh = plsc.VectorSubcoreMesh(core_axis_name="c", subcore_axis_name="s")


@functools.partial(
    pl.kernel,
    mesh=_mesh,
    out_type=jax.ShapeDtypeStruct((NC * ACC,), jnp.float32),
    scratch_types=[
        pltpu.VMEM_SHARED((ACC,), jnp.float32),
        pltpu.VMEM((ZB,), jnp.float32),
        pltpu.VMEM((CHUNK,), jnp.int32),
        pltpu.VMEM((CHUNK,), jnp.int32),
        pltpu.VMEM((CHUNK,), jnp.float32),
    ],
)
def _sc_scatter(v0, v1, v2, v3, v4, v5, idx_hbm, part_hbm,
                acc, zb, ir, ish, vw):
    c = lax.axis_index("c")
    s = lax.axis_index("s")
    tile = c * NS + s

    @pl.loop(0, ZB // 16)
    def _(i):
        zb[pl.ds(i * 16, 16)] = jnp.zeros((16,), jnp.float32)

    a0 = s * ACC_SLICE

    @pl.loop(0, ACC_SLICE // ZB)
    def _(i):
        pltpu.sync_copy(zb, acc.at[pl.ds(a0 + i * ZB, ZB)])

    base = tile * CHUNK
    pltpu.sync_copy(idx_hbm.at[pl.ds(base, CHUNK)], ir)
    plsc.subcore_barrier()

    pltpu.sync_copy(v0.at[pl.ds(base, CHUNK)], vw)
    pltpu.sync_copy(vw, acc.at[ir], add=True)

    for f, vf in enumerate((v1, v2, v3, v4, v5), start=1):
        src = ir if f == 1 else ish

        @pl.loop(0, CHUNK // 16)
        def _(i, src=src):
            sl = pl.ds(i * 16, 16)
            ish[sl] = src[sl] + NCELLS

        pltpu.sync_copy(vf.at[pl.ds(base, CHUNK)], vw)
        pltpu.sync_copy(vw, acc.at[ish], add=True)

    plsc.subcore_barrier()
    pltpu.sync_copy(acc.at[pl.ds(a0, ACC_SLICE)],
                    part_hbm.at[pl.ds(c * ACC + a0, ACC_SLICE)])


def _tc_sum_body(p_ref, o_ref):
    p = p_ref[...]
    o_ref[...] = p[0] + p[1]


_TCB = 1536

_tc_sum = pl.pallas_call(
    _tc_sum_body,
    grid=(ACC // 128 // _TCB,),
    in_specs=[pl.BlockSpec((NC, _TCB, 128), lambda i: (0, i, 0))],
    out_specs=pl.BlockSpec((_TCB, 128), lambda i: (i, 0)),
    out_shape=jax.ShapeDtypeStruct((ACC // 128, 128), jnp.float32),
)


def kernel(x, indices):
    n = x.shape[0]
    idx = indices.astype(jnp.int32)
    npad = NP_PAD - n
    idx_pad = jnp.concatenate(
        [idx, (jnp.arange(npad, dtype=jnp.int32) * 97) % NCELLS])
    xf = x.astype(jnp.float32)
    zpad = jnp.zeros((npad,), jnp.float32)
    vfs = [jnp.concatenate([xf[:, f], zpad]) for f in range(F)]
    part = _sc_scatter(*vfs, idx_pad)
    grid2d = _tc_sum(part.reshape(NC, ACC // 128, 128))
    return grid2d.reshape(F, NX, NY)


# speedup vs baseline: 7.8259x; 1.0790x over previous
import functools

import jax
import jax.numpy as jnp
from jax import lax
from jax.experimental import pallas as pl
from jax.experimental.pallas import tpu as pltpu
from jax.experimental.pallas import tpu_sc as plsc

NX = 512
NY = 512
NCELLS = NX * NY
F = 6
NC = 2
NS = 16
NT = NC * NS
CHUNK = 6272
NP_PAD = NT * CHUNK
ACC = F * NCELLS
ACC_SLICE = ACC // NS
ZB = 4096

_mesh = plsc.VectorSubcoreMesh(core_axis_name="c", subcore_axis_name="s")


@functools.partial(
    pl.kernel,
    mesh=_mesh,
    out_type=jax.ShapeDtypeStruct((NC * ACC,), jnp.float32),
    scratch_types=[
        pltpu.VMEM_SHARED((ACC,), jnp.float32),
        pltpu.VMEM((ZB,), jnp.float32),
        pltpu.VMEM((CHUNK,), jnp.int32),
        pltpu.VMEM((CHUNK,), jnp.int32),
        pltpu.VMEM((CHUNK,), jnp.float32),
        pltpu.VMEM((CHUNK,), jnp.float32),
        pltpu.SemaphoreType.DMA,
        pltpu.SemaphoreType.DMA,
    ],
)
def _sc_scatter(v0, v1, v2, v3, v4, v5, idx_hbm, part_hbm,
                acc, zb, isha, ishb, vwa, vwb, sla, slb):
    c = lax.axis_index("c")
    s = lax.axis_index("s")
    tile = c * NS + s
    base = tile * CHUNK
    a0 = s * ACC_SLICE
    vfs = (v0, v1, v2, v3, v4, v5)
    bufs = (vwa, vwb)
    ishs = (isha, ishb)
    lsems = (sla, slb)

    idx_load = pltpu.async_copy(idx_hbm.at[pl.ds(base, CHUNK)], isha, slb)
    loads = [pltpu.async_copy(v0.at[pl.ds(base, CHUNK)], vwa, sla)]

    @pl.loop(0, ZB // 16)
    def _(i):
        zb[pl.ds(i * 16, 16)] = jnp.zeros((16,), jnp.float32)

    @pl.loop(0, ACC_SLICE // ZB)
    def _(i):
        pltpu.sync_copy(zb, acc.at[pl.ds(a0 + i * ZB, ZB)])

    idx_load.wait()
    plsc.subcore_barrier()

    for f in range(F):
        b = f % 2
        if f + 1 < F:
            loads.append(
                pltpu.async_copy(vfs[f + 1].at[pl.ds(base, CHUNK)],
                                 bufs[1 - b], lsems[1 - b]))

            @pl.loop(0, CHUNK // 16)
            def _(i, b=b):
                sl = pl.ds(i * 16, 16)
                ishs[1 - b][sl] = ishs[b][sl] + NCELLS

        loads[f].wait()
        pltpu.sync_copy(bufs[b], acc.at[ishs[b]], add=True)

    plsc.subcore_barrier()
    pltpu.sync_copy(acc.at[pl.ds(a0, ACC_SLICE)],
                    part_hbm.at[pl.ds(c * ACC + a0, ACC_SLICE)])


def _tc_sum_body(p_ref, o_ref):
    p = p_ref[...]
    o_ref[...] = p[0] + p[1]


_TCB = 1536

_tc_sum = pl.pallas_call(
    _tc_sum_body,
    grid=(ACC // 128 // _TCB,),
    in_specs=[pl.BlockSpec((NC, _TCB, 128), lambda i: (0, i, 0))],
    out_specs=pl.BlockSpec((_TCB, 128), lambda i: (i, 0)),
    out_shape=jax.ShapeDtypeStruct((ACC // 128, 128), jnp.float32),
)


def kernel(x, indices):
    n = x.shape[0]
    idx = indices.astype(jnp.int32)
    npad = NP_PAD - n
    idx_pad = jnp.concatenate(
        [idx, (jnp.arange(npad, dtype=jnp.int32) * 97) % NCELLS])
    xf = x.astype(jnp.float32)
    zpad = jnp.zeros((npad,), jnp.float32)
    vfs = [jnp.concatenate([xf[:, f], zpad]) for f in range(F)]
    part = _sc_scatter(*vfs, idx_pad)
    grid2d = _tc_sum(part.reshape(NC, ACC // 128, 128))
    return grid2d.reshape(F, NX, NY)
